# Initial kernel scaffold; baseline (speedup 1.0000x reference)
import jax
import jax.numpy as jnp
from jax.experimental import pallas as pl


def kernel(x, edge_index, batch, W1, b1, W2, b2, W3, b3, Wout, bout):
    raise NotImplementedError("write your pallas kernel here")


# speedup vs baseline: 6.5972x; 6.5972x over previous
import functools

import jax
import jax.numpy as jnp
from jax import lax
from jax.experimental import pallas as pl
from jax.experimental.pallas import tpu as pltpu
from jax.experimental.pallas import tpu_sc as plsc

N = 10000
E = 320000
B = 64
NSC = 2
NTIL = 16
NB = 125
K = 80
DRT = 624
TOFF = DRT * NTIL
TAIL = N - TOFF
ZR = 208
NP1 = 10240

@functools.cache
def _make_deg():
    return pl.kernel(
        _deg_body,
        out_type=jax.ShapeDtypeStruct((NSC * NP1,), jnp.float32),
        mesh=plsc.VectorSubcoreMesh(core_axis_name="c", subcore_axis_name="s"),
        scratch_types=[
            pltpu.VMEM((NB, K), jnp.int32),
            pltpu.VMEM((NB, K), jnp.float32),
            pltpu.VMEM((1024,), jnp.float32),
            pltpu.VMEM_SHARED((NP1,), jnp.float32),
        ],
    )


def _deg_body(dst_hbm, ones_hbm, z1_hbm, out_hbm, idx_v, ones_v, stage, acc):
    c = lax.axis_index("c")
    s = lax.axis_index("s")
    pltpu.sync_copy(dst_hbm.at[c, s], idx_v)
    pltpu.sync_copy(ones_hbm, ones_v)

    @pl.when(s < 10)
    def _():
        pltpu.sync_copy(z1_hbm, stage)
        pltpu.sync_copy(stage, acc.at[pl.ds(s * 1024, 1024)])

    plsc.subcore_barrier()

    def body(j, carry):
        pltpu.sync_copy(ones_v.at[j], acc.at[idx_v.at[j]], add=True)
        return carry

    lax.fori_loop(0, NB, body, 0)
    plsc.subcore_barrier()

    @pl.when(s < 10)
    def _():
        pltpu.sync_copy(acc.at[pl.ds(s * 1024, 1024)], stage)
        pltpu.sync_copy(stage, out_hbm.at[pl.ds(c * NP1 + s * 1024, 1024)])


@functools.cache
def _make_agg(w, nch):
    def body(*refs):
        h_list = refs[:nch]
        src_hbm, dst_hbm, z2_hbm = refs[nch:nch + 3]
        out_list = refs[nch + 3:2 * nch + 3]
        src_v, dst_v, buf, zbuf, obuf, acc, sem = refs[2 * nch + 3:]
        c = lax.axis_index("c")
        s = lax.axis_index("s")
        pltpu.sync_copy(src_hbm.at[c, s], src_v)
        pltpu.sync_copy(dst_hbm.at[c, s], dst_v)
        pltpu.sync_copy(z2_hbm, zbuf)
        for ch in range(nch):
            for k3 in range(3):
                pltpu.sync_copy(zbuf, acc.at[pl.ds(s * DRT + k3 * ZR, ZR)])

            @pl.when(s == NTIL - 1)
            def _():
                pltpu.sync_copy(zbuf.at[pl.ds(0, TAIL)],
                                acc.at[pl.ds(TOFF, TAIL)])

            plsc.subcore_barrier()

            def ebody(j, carry):
                pltpu.async_copy(h_list[ch].at[src_v.at[j]], buf, sem).wait()
                pltpu.sync_copy(buf, acc.at[dst_v.at[j]], add=True)
                return carry

            lax.fori_loop(0, NB, ebody, 0)
            plsc.subcore_barrier()
            for k3 in range(3):
                pltpu.sync_copy(acc.at[pl.ds(s * DRT + k3 * ZR, ZR)], obuf)
                pltpu.sync_copy(
                    obuf, out_list[ch].at[c, pl.ds(s * DRT + k3 * ZR, ZR)])

            @pl.when(s == NTIL - 1)
            def _():
                pltpu.sync_copy(acc.at[pl.ds(TOFF, TAIL)],
                                obuf.at[pl.ds(0, TAIL)])
                pltpu.sync_copy(obuf.at[pl.ds(0, TAIL)],
                                out_list[ch].at[c, pl.ds(TOFF, TAIL)])

    return pl.kernel(
        body,
        out_type=[jax.ShapeDtypeStruct((NSC, N, w), jnp.float32)] * nch,
        mesh=plsc.VectorSubcoreMesh(core_axis_name="c", subcore_axis_name="s"),
        compiler_params=pltpu.CompilerParams(use_tc_tiling_on_sc=False),
        scratch_types=[
            pltpu.VMEM((NB, K), jnp.int32),
            pltpu.VMEM((NB, K), jnp.int32),
            pltpu.VMEM((K, w), jnp.float32),
            pltpu.VMEM((ZR, w), jnp.float32),
            pltpu.VMEM((ZR, w), jnp.float32),
            pltpu.VMEM_SHARED((N, w), jnp.float32),
            pltpu.SemaphoreType.DMA,
        ],
    )


def _scale_body(dp_ref, x_ref, xs0_ref, xs1_ref, dis_ref):
    deg = dp_ref[...][0] + dp_ref[...][1] + 1.0
    dis = lax.rsqrt(deg)
    dis_ref[...] = jnp.broadcast_to(dis, (N, 128))
    xs = x_ref[...] * dis
    xs0_ref[...] = xs[:, :64]
    xs1_ref[...] = xs[:, 64:]


_scale = pl.pallas_call(
    _scale_body,
    out_shape=(jax.ShapeDtypeStruct((N, 64), jnp.float32),
               jax.ShapeDtypeStruct((N, 64), jnp.float32),
               jax.ShapeDtypeStruct((N, 128), jnp.float32)),
)


def _make_layer(nch_in, w_in, nch_out, w_out, d_out, scale_out, R=1000):
    d_in = nch_in * w_in

    def body(*refs):
        agg_refs = refs[0:nch_in]
        xs_refs = refs[nch_in:2 * nch_in]
        dis_ref = refs[2 * nch_in]
        w_ref = refs[1 + 2 * nch_in]
        b_ref = refs[2 + 2 * nch_in]
        out_refs = refs[3 + 2 * nch_in:]
        aggc = jnp.concatenate(
            [r[...][0] + r[...][1] for r in agg_refs], axis=1)
        xsc = jnp.concatenate([r[...] for r in xs_refs], axis=1)
        dis = dis_ref[...][:, 0:1]
        g = dis * (aggc + xsc)
        h = lax.dot_general(g, w_ref[...], (((1,), (1,)), ((), ())),
                            preferred_element_type=jnp.float32)
        h = jnp.maximum(h + b_ref[...], 0.0)
        if scale_out:
            h = h * dis
        for k2 in range(nch_out):
            out_refs[k2][...] = h[:, k2 * w_out:(k2 + 1) * w_out]

    return pl.pallas_call(
        body,
        grid=(N // R,),
        in_specs=(
            [pl.BlockSpec((NSC, R, w_in), lambda i: (0, i, 0))] * nch_in
            + [pl.BlockSpec((R, w_in), lambda i: (i, 0))] * nch_in
            + [pl.BlockSpec((R, 128), lambda i: (i, 0)),
               pl.BlockSpec((d_out, d_in), lambda i: (0, 0)),
               pl.BlockSpec((1, d_out), lambda i: (0, 0))]),
        out_specs=[pl.BlockSpec((R, w_out), lambda i: (i, 0))] * nch_out,
        out_shape=[jax.ShapeDtypeStruct((N, w_out), jnp.float32)] * nch_out,
    )


_layer1 = _make_layer(2, 64, 4, 64, 256, True)
_layer2 = _make_layer(4, 64, 16, 32, 512, True)
_layer3 = _make_layer(16, 32, 8, 128, 1024, False, R=400)


def _pool_body(*refs):
    b_ref, w_ref, bo_ref = refs[0], refs[1], refs[2]
    h_refs = refs[3:11]
    out_ref = refs[11]
    h3 = jnp.concatenate([r[...] for r in h_refs], axis=1)
    bvec = b_ref[...]
    io = lax.broadcasted_iota(jnp.int32, (B, N), 0)
    oh = (bvec == io).astype(jnp.float32)
    pooled = lax.dot_general(oh, h3, (((1,), (0,)), ((), ())),
                             preferred_element_type=jnp.float32)
    cnt = jnp.sum(oh, axis=1, keepdims=True)
    pooled = pooled / jnp.maximum(cnt, 1.0)
    out = lax.dot_general(pooled, w_ref[...], (((1,), (1,)), ((), ())),
                          preferred_element_type=jnp.float32)
    out_ref[...] = out + bo_ref[...]


_pool = pl.pallas_call(
    _pool_body,
    out_shape=jax.ShapeDtypeStruct((B, 1024), jnp.float32),
)


def kernel(x, edge_index, batch, W1, b1, W2, b2, W3, b3, Wout, bout):
    src_r = edge_index[0].reshape(NSC, NTIL, NB, K)
    dst_r = edge_index[1].reshape(NSC, NTIL, NB, K)
    ones_upd = jnp.ones((NB, K), jnp.float32)
    z1 = jnp.zeros((1024,), jnp.float32)
    z64 = jnp.zeros((ZR, 64), jnp.float32)
    z32 = jnp.zeros((ZR, 32), jnp.float32)

    degp = _make_deg()(dst_r, ones_upd, z1)
    degp = degp.reshape(NSC, NP1)[:, :N].reshape(NSC, N, 1)
    xs0a, xs0b, dis2d = _scale(degp, x)
    a1 = _make_agg(64, 2)(xs0a, xs0b, src_r, dst_r, z64)
    xs1 = _layer1(*a1, xs0a, xs0b, dis2d, W1, b1.reshape(1, -1))
    a2 = _make_agg(64, 4)(*xs1, src_r, dst_r, z64)
    xs2 = _layer2(*a2, *xs1, dis2d, W2, b2.reshape(1, -1))
    a3 = _make_agg(32, 16)(*xs2, src_r, dst_r, z32)
    h3 = _layer3(*a3, *xs2, dis2d, W3, b3.reshape(1, -1))
    return _pool(batch.reshape(1, N), Wout, bout.reshape(1, -1), *h3)


# speedup vs baseline: 14.8736x; 2.2545x over previous
import functools

import jax
import jax.numpy as jnp
from jax import lax
from jax.experimental import pallas as pl
from jax.experimental.pallas import tpu as pltpu
from jax.experimental.pallas import tpu_sc as plsc

N = 10000
E = 320000
B = 64
NSC = 2
NTIL = 16
NB = 100
K = 100
NBUF = 4
DRT = 624
TOFF = DRT * NTIL
TAIL = N - TOFF
ZR = 208
NP1 = 10240

@functools.cache
def _make_deg():
    return pl.kernel(
        _deg_body,
        out_type=jax.ShapeDtypeStruct((NSC * NP1,), jnp.float32),
        mesh=plsc.VectorSubcoreMesh(core_axis_name="c", subcore_axis_name="s"),
        scratch_types=[
            pltpu.VMEM((NB, K), jnp.int32),
            pltpu.VMEM((NB, K), jnp.float32),
            pltpu.VMEM((1024,), jnp.float32),
            pltpu.VMEM_SHARED((NP1,), jnp.float32),
        ],
    )


def _deg_body(dst_hbm, ones_hbm, z1_hbm, out_hbm, idx_v, ones_v, stage, acc):
    c = lax.axis_index("c")
    s = lax.axis_index("s")
    pltpu.sync_copy(dst_hbm.at[c, s], idx_v)
    pltpu.sync_copy(ones_hbm, ones_v)

    @pl.when(s < 10)
    def _():
        pltpu.sync_copy(z1_hbm, stage)
        pltpu.sync_copy(stage, acc.at[pl.ds(s * 1024, 1024)])

    plsc.subcore_barrier()

    def body(j, carry):
        pltpu.sync_copy(ones_v.at[j], acc.at[idx_v.at[j]], add=True)
        return carry

    lax.fori_loop(0, NB, body, 0)
    plsc.subcore_barrier()

    @pl.when(s < 10)
    def _():
        pltpu.sync_copy(acc.at[pl.ds(s * 1024, 1024)], stage)
        pltpu.sync_copy(stage, out_hbm.at[pl.ds(c * NP1 + s * 1024, 1024)])


@functools.cache
def _make_agg(w, nch):
    def body(*refs):
        h_list = refs[:nch]
        src_hbm, dst_hbm, z2_hbm = refs[nch:nch + 3]
        out_list = refs[nch + 3:2 * nch + 3]
        (src_v, dst_v, zbuf, obuf, acc) = refs[2 * nch + 3:2 * nch + 8]
        bufs = refs[2 * nch + 8:2 * nch + 8 + NBUF]
        gsem = refs[2 * nch + 8 + NBUF:2 * nch + 8 + 2 * NBUF]
        ssem = refs[2 * nch + 8 + 2 * NBUF:]
        c = lax.axis_index("c")
        s = lax.axis_index("s")
        pltpu.sync_copy(src_hbm.at[c, s], src_v)
        pltpu.sync_copy(dst_hbm.at[c, s], dst_v)
        pltpu.sync_copy(z2_hbm, zbuf)
        for ch in range(nch):
            h_hbm = h_list[ch]
            for k3 in range(3):
                pltpu.sync_copy(zbuf, acc.at[pl.ds(s * DRT + k3 * ZR, ZR)])

            @pl.when(s == NTIL - 1)
            def _():
                pltpu.sync_copy(zbuf.at[pl.ds(0, TAIL)],
                                acc.at[pl.ds(TOFF, TAIL)])

            plsc.subcore_barrier()

            def ebody(t, carry):
                base = t * NBUF
                for b in range(NBUF):
                    @pl.when(t > 0)
                    def _():
                        pltpu.make_async_copy(
                            bufs[b], acc.at[dst_v.at[base - NBUF + b]],
                            ssem[b]).wait()
                    pltpu.async_copy(h_hbm.at[src_v.at[base + b]],
                                     bufs[b], gsem[b])
                for b in range(NBUF):
                    pltpu.make_async_copy(h_hbm.at[src_v.at[base + b]],
                                          bufs[b], gsem[b]).wait()
                    pltpu.async_copy(bufs[b], acc.at[dst_v.at[base + b]],
                                     ssem[b], add=True)
                return carry

            lax.fori_loop(0, NB // NBUF, ebody, 0)
            for b in range(NBUF):
                pltpu.make_async_copy(
                    bufs[b], acc.at[dst_v.at[NB - NBUF + b]], ssem[b]).wait()
            plsc.subcore_barrier()
            for k3 in range(3):
                pltpu.sync_copy(acc.at[pl.ds(s * DRT + k3 * ZR, ZR)], obuf)
                pltpu.sync_copy(
                    obuf, out_list[ch].at[c, pl.ds(s * DRT + k3 * ZR, ZR)])

            @pl.when(s == NTIL - 1)
            def _():
                pltpu.sync_copy(acc.at[pl.ds(TOFF, TAIL)],
                                obuf.at[pl.ds(0, TAIL)])
                pltpu.sync_copy(obuf.at[pl.ds(0, TAIL)],
                                out_list[ch].at[c, pl.ds(TOFF, TAIL)])

    return pl.kernel(
        body,
        out_type=[jax.ShapeDtypeStruct((NSC, N, w), jnp.float32)] * nch,
        mesh=plsc.VectorSubcoreMesh(core_axis_name="c", subcore_axis_name="s"),
        compiler_params=pltpu.CompilerParams(use_tc_tiling_on_sc=False),
        scratch_types=(
            [pltpu.VMEM((NB, K), jnp.int32),
             pltpu.VMEM((NB, K), jnp.int32),
             pltpu.VMEM((ZR, w), jnp.float32),
             pltpu.VMEM((ZR, w), jnp.float32),
             pltpu.VMEM_SHARED((N, w), jnp.float32)]
            + [pltpu.VMEM((K, w), jnp.float32)] * NBUF
            + [pltpu.SemaphoreType.DMA] * (2 * NBUF)
        ),
    )


def _scale_body(dp_ref, x_ref, xs0_ref, xs1_ref, dis_ref):
    deg = dp_ref[...][0] + dp_ref[...][1] + 1.0
    dis = lax.rsqrt(deg)
    dis_ref[...] = jnp.broadcast_to(dis, (N, 128))
    xs = x_ref[...] * dis
    xs0_ref[...] = xs[:, :64]
    xs1_ref[...] = xs[:, 64:]


_scale = pl.pallas_call(
    _scale_body,
    out_shape=(jax.ShapeDtypeStruct((N, 64), jnp.float32),
               jax.ShapeDtypeStruct((N, 64), jnp.float32),
               jax.ShapeDtypeStruct((N, 128), jnp.float32)),
)


def _make_layer(nch_in, w_in, nch_out, w_out, d_out, scale_out, R=1000):
    d_in = nch_in * w_in

    def body(*refs):
        agg_refs = refs[0:nch_in]
        xs_refs = refs[nch_in:2 * nch_in]
        dis_ref = refs[2 * nch_in]
        w_ref = refs[1 + 2 * nch_in]
        b_ref = refs[2 + 2 * nch_in]
        out_refs = refs[3 + 2 * nch_in:]
        aggc = jnp.concatenate(
            [r[...][0] + r[...][1] for r in agg_refs], axis=1)
        xsc = jnp.concatenate([r[...] for r in xs_refs], axis=1)
        dis = dis_ref[...][:, 0:1]
        g = dis * (aggc + xsc)
        h = lax.dot_general(g, w_ref[...], (((1,), (1,)), ((), ())),
                            preferred_element_type=jnp.float32)
        h = jnp.maximum(h + b_ref[...], 0.0)
        if scale_out:
            h = h * dis
        for k2 in range(nch_out):
            out_refs[k2][...] = h[:, k2 * w_out:(k2 + 1) * w_out]

    return pl.pallas_call(
        body,
        grid=(N // R,),
        in_specs=(
            [pl.BlockSpec((NSC, R, w_in), lambda i: (0, i, 0))] * nch_in
            + [pl.BlockSpec((R, w_in), lambda i: (i, 0))] * nch_in
            + [pl.BlockSpec((R, 128), lambda i: (i, 0)),
               pl.BlockSpec((d_out, d_in), lambda i: (0, 0)),
               pl.BlockSpec((1, d_out), lambda i: (0, 0))]),
        out_specs=[pl.BlockSpec((R, w_out), lambda i: (i, 0))] * nch_out,
        out_shape=[jax.ShapeDtypeStruct((N, w_out), jnp.float32)] * nch_out,
    )


_layer1 = _make_layer(2, 64, 4, 64, 256, True)
_layer2 = _make_layer(4, 64, 16, 32, 512, True)
_layer3 = _make_layer(16, 32, 8, 128, 1024, False, R=400)


def _pool_body(*refs):
    b_ref, w_ref, bo_ref = refs[0], refs[1], refs[2]
    h_refs = refs[3:11]
    out_ref = refs[11]
    h3 = jnp.concatenate([r[...] for r in h_refs], axis=1)
    bvec = b_ref[...]
    io = lax.broadcasted_iota(jnp.int32, (B, N), 0)
    oh = (bvec == io).astype(jnp.float32)
    pooled = lax.dot_general(oh, h3, (((1,), (0,)), ((), ())),
                             preferred_element_type=jnp.float32)
    cnt = jnp.sum(oh, axis=1, keepdims=True)
    pooled = pooled / jnp.maximum(cnt, 1.0)
    out = lax.dot_general(pooled, w_ref[...], (((1,), (1,)), ((), ())),
                          preferred_element_type=jnp.float32)
    out_ref[...] = out + bo_ref[...]


_pool = pl.pallas_call(
    _pool_body,
    out_shape=jax.ShapeDtypeStruct((B, 1024), jnp.float32),
)


def kernel(x, edge_index, batch, W1, b1, W2, b2, W3, b3, Wout, bout):
    src_r = edge_index[0].reshape(NSC, NTIL, NB, K)
    dst_r = edge_index[1].reshape(NSC, NTIL, NB, K)
    ones_upd = jnp.ones((NB, K), jnp.float32)
    z1 = jnp.zeros((1024,), jnp.float32)
    z64 = jnp.zeros((ZR, 64), jnp.float32)
    z32 = jnp.zeros((ZR, 32), jnp.float32)

    degp = _make_deg()(dst_r, ones_upd, z1)
    degp = degp.reshape(NSC, NP1)[:, :N].reshape(NSC, N, 1)
    xs0a, xs0b, dis2d = _scale(degp, x)
    a1 = _make_agg(64, 2)(xs0a, xs0b, src_r, dst_r, z64)
    xs1 = _layer1(*a1, xs0a, xs0b, dis2d, W1, b1.reshape(1, -1))
    a2 = _make_agg(64, 4)(*xs1, src_r, dst_r, z64)
    xs2 = _layer2(*a2, *xs1, dis2d, W2, b2.reshape(1, -1))
    a3 = _make_agg(32, 16)(*xs2, src_r, dst_r, z32)
    h3 = _layer3(*a3, *xs2, dis2d, W3, b3.reshape(1, -1))
    return _pool(batch.reshape(1, N), Wout, bout.reshape(1, -1), *h3)


# speedup vs baseline: 15.3607x; 1.0327x over previous
import functools

import jax
import jax.numpy as jnp
from jax import lax
from jax.experimental import pallas as pl
from jax.experimental.pallas import tpu as pltpu
from jax.experimental.pallas import tpu_sc as plsc

N = 10000
E = 320000
B = 64
NSC = 2
NTIL = 16
NB = 80
K = 125
NBUF = 4
DRT = 624
TOFF = DRT * NTIL
TAIL = N - TOFF
ZR = 208
NP1 = 10240

@functools.cache
def _make_deg():
    return pl.kernel(
        _deg_body,
        out_type=jax.ShapeDtypeStruct((NSC * NP1,), jnp.float32),
        mesh=plsc.VectorSubcoreMesh(core_axis_name="c", subcore_axis_name="s"),
        scratch_types=[
            pltpu.VMEM((NB, K), jnp.int32),
            pltpu.VMEM((NB, K), jnp.float32),
            pltpu.VMEM((1024,), jnp.float32),
            pltpu.VMEM_SHARED((NP1,), jnp.float32),
        ],
    )


def _deg_body(dst_hbm, ones_hbm, z1_hbm, out_hbm, idx_v, ones_v, stage, acc):
    c = lax.axis_index("c")
    s = lax.axis_index("s")
    pltpu.sync_copy(dst_hbm.at[c, s], idx_v)
    pltpu.sync_copy(ones_hbm, ones_v)

    @pl.when(s < 10)
    def _():
        pltpu.sync_copy(z1_hbm, stage)
        pltpu.sync_copy(stage, acc.at[pl.ds(s * 1024, 1024)])

    plsc.subcore_barrier()

    def body(j, carry):
        pltpu.sync_copy(ones_v.at[j], acc.at[idx_v.at[j]], add=True)
        return carry

    lax.fori_loop(0, NB, body, 0)
    plsc.subcore_barrier()

    @pl.when(s < 10)
    def _():
        pltpu.sync_copy(acc.at[pl.ds(s * 1024, 1024)], stage)
        pltpu.sync_copy(stage, out_hbm.at[pl.ds(c * NP1 + s * 1024, 1024)])


@functools.cache
def _make_agg(w, nch):
    def body(*refs):
        h_list = refs[:nch]
        src_hbm, dst_hbm, z2_hbm = refs[nch:nch + 3]
        out_list = refs[nch + 3:2 * nch + 3]
        (src_v, dst_v, zbuf, obuf, acc) = refs[2 * nch + 3:2 * nch + 8]
        bufs = refs[2 * nch + 8:2 * nch + 8 + NBUF]
        gsem = refs[2 * nch + 8 + NBUF:2 * nch + 8 + 2 * NBUF]
        ssem = refs[2 * nch + 8 + 2 * NBUF:]
        c = lax.axis_index("c")
        s = lax.axis_index("s")
        pltpu.sync_copy(src_hbm.at[c, s], src_v)
        pltpu.sync_copy(dst_hbm.at[c, s], dst_v)
        pltpu.sync_copy(z2_hbm, zbuf)
        for ch in range(nch):
            h_hbm = h_list[ch]
            for k3 in range(3):
                pltpu.sync_copy(zbuf, acc.at[pl.ds(s * DRT + k3 * ZR, ZR)])

            @pl.when(s == NTIL - 1)
            def _():
                pltpu.sync_copy(zbuf.at[pl.ds(0, TAIL)],
                                acc.at[pl.ds(TOFF, TAIL)])

            plsc.subcore_barrier()

            def ebody(t, carry):
                base = t * NBUF
                for b in range(NBUF):
                    @pl.when(t > 0)
                    def _():
                        pltpu.make_async_copy(
                            bufs[b], acc.at[dst_v.at[base - NBUF + b]],
                            ssem[b]).wait()
                    pltpu.async_copy(h_hbm.at[src_v.at[base + b]],
                                     bufs[b], gsem[b])
                for b in range(NBUF):
                    pltpu.make_async_copy(h_hbm.at[src_v.at[base + b]],
                                          bufs[b], gsem[b]).wait()
                    pltpu.async_copy(bufs[b], acc.at[dst_v.at[base + b]],
                                     ssem[b], add=True)
                return carry

            lax.fori_loop(0, NB // NBUF, ebody, 0)
            for b in range(NBUF):
                pltpu.make_async_copy(
                    bufs[b], acc.at[dst_v.at[NB - NBUF + b]], ssem[b]).wait()
            plsc.subcore_barrier()
            for k3 in range(3):
                pltpu.sync_copy(acc.at[pl.ds(s * DRT + k3 * ZR, ZR)], obuf)
                pltpu.sync_copy(
                    obuf, out_list[ch].at[c, pl.ds(s * DRT + k3 * ZR, ZR)])

            @pl.when(s == NTIL - 1)
            def _():
                pltpu.sync_copy(acc.at[pl.ds(TOFF, TAIL)],
                                obuf.at[pl.ds(0, TAIL)])
                pltpu.sync_copy(obuf.at[pl.ds(0, TAIL)],
                                out_list[ch].at[c, pl.ds(TOFF, TAIL)])

    return pl.kernel(
        body,
        out_type=[jax.ShapeDtypeStruct((NSC, N, w), jnp.float32)] * nch,
        mesh=plsc.VectorSubcoreMesh(core_axis_name="c", subcore_axis_name="s"),
        compiler_params=pltpu.CompilerParams(use_tc_tiling_on_sc=False),
        scratch_types=(
            [pltpu.VMEM((NB, K), jnp.int32),
             pltpu.VMEM((NB, K), jnp.int32),
             pltpu.VMEM((ZR, w), jnp.float32),
             pltpu.VMEM((ZR, w), jnp.float32),
             pltpu.VMEM_SHARED((N, w), jnp.float32)]
            + [pltpu.VMEM((K, w), jnp.float32)] * NBUF
            + [pltpu.SemaphoreType.DMA] * (2 * NBUF)
        ),
    )


def _scale_body(dp_ref, x_ref, xs0_ref, xs1_ref, dis_ref):
    deg = dp_ref[...][0] + dp_ref[...][1] + 1.0
    dis = lax.rsqrt(deg)
    dis_ref[...] = jnp.broadcast_to(dis, (N, 128))
    xs = x_ref[...] * dis
    xs0_ref[...] = xs[:, :64]
    xs1_ref[...] = xs[:, 64:]


_scale = pl.pallas_call(
    _scale_body,
    out_shape=(jax.ShapeDtypeStruct((N, 64), jnp.float32),
               jax.ShapeDtypeStruct((N, 64), jnp.float32),
               jax.ShapeDtypeStruct((N, 128), jnp.float32)),
)


def _make_layer(nch_in, w_in, nch_out, w_out, d_out, scale_out, R=1000):
    d_in = nch_in * w_in

    def body(*refs):
        agg_refs = refs[0:nch_in]
        xs_refs = refs[nch_in:2 * nch_in]
        dis_ref = refs[2 * nch_in]
        w_ref = refs[1 + 2 * nch_in]
        b_ref = refs[2 + 2 * nch_in]
        out_refs = refs[3 + 2 * nch_in:]
        aggc = jnp.concatenate(
            [r[...][0] + r[...][1] for r in agg_refs], axis=1)
        xsc = jnp.concatenate([r[...] for r in xs_refs], axis=1)
        dis = dis_ref[...][:, 0:1]
        g = dis * (aggc + xsc)
        h = lax.dot_general(g, w_ref[...], (((1,), (1,)), ((), ())),
                            preferred_element_type=jnp.float32)
        h = jnp.maximum(h + b_ref[...], 0.0)
        if scale_out:
            h = h * dis
        for k2 in range(nch_out):
            out_refs[k2][...] = h[:, k2 * w_out:(k2 + 1) * w_out]

    return pl.pallas_call(
        body,
        grid=(N // R,),
        in_specs=(
            [pl.BlockSpec((NSC, R, w_in), lambda i: (0, i, 0))] * nch_in
            + [pl.BlockSpec((R, w_in), lambda i: (i, 0))] * nch_in
            + [pl.BlockSpec((R, 128), lambda i: (i, 0)),
               pl.BlockSpec((d_out, d_in), lambda i: (0, 0)),
               pl.BlockSpec((1, d_out), lambda i: (0, 0))]),
        out_specs=[pl.BlockSpec((R, w_out), lambda i: (i, 0))] * nch_out,
        out_shape=[jax.ShapeDtypeStruct((N, w_out), jnp.float32)] * nch_out,
    )


_layer1 = _make_layer(2, 64, 4, 64, 256, True)
_layer2 = _make_layer(4, 64, 16, 32, 512, True)
_layer3 = _make_layer(16, 32, 8, 128, 1024, False, R=400)


def _pool_body(*refs):
    b_ref, w_ref, bo_ref = refs[0], refs[1], refs[2]
    h_refs = refs[3:11]
    out_ref = refs[11]
    h3 = jnp.concatenate([r[...] for r in h_refs], axis=1)
    bvec = b_ref[...]
    io = lax.broadcasted_iota(jnp.int32, (B, N), 0)
    oh = (bvec == io).astype(jnp.float32)
    pooled = lax.dot_general(oh, h3, (((1,), (0,)), ((), ())),
                             preferred_element_type=jnp.float32)
    cnt = jnp.sum(oh, axis=1, keepdims=True)
    pooled = pooled / jnp.maximum(cnt, 1.0)
    out = lax.dot_general(pooled, w_ref[...], (((1,), (1,)), ((), ())),
                          preferred_element_type=jnp.float32)
    out_ref[...] = out + bo_ref[...]


_pool = pl.pallas_call(
    _pool_body,
    out_shape=jax.ShapeDtypeStruct((B, 1024), jnp.float32),
)


def kernel(x, edge_index, batch, W1, b1, W2, b2, W3, b3, Wout, bout):
    src_r = edge_index[0].reshape(NSC, NTIL, NB, K)
    dst_r = edge_index[1].reshape(NSC, NTIL, NB, K)
    ones_upd = jnp.ones((NB, K), jnp.float32)
    z1 = jnp.zeros((1024,), jnp.float32)
    z64 = jnp.zeros((ZR, 64), jnp.float32)
    z32 = jnp.zeros((ZR, 32), jnp.float32)

    degp = _make_deg()(dst_r, ones_upd, z1)
    degp = degp.reshape(NSC, NP1)[:, :N].reshape(NSC, N, 1)
    xs0a, xs0b, dis2d = _scale(degp, x)
    a1 = _make_agg(64, 2)(xs0a, xs0b, src_r, dst_r, z64)
    xs1 = _layer1(*a1, xs0a, xs0b, dis2d, W1, b1.reshape(1, -1))
    a2 = _make_agg(64, 4)(*xs1, src_r, dst_r, z64)
    xs2 = _layer2(*a2, *xs1, dis2d, W2, b2.reshape(1, -1))
    a3 = _make_agg(32, 16)(*xs2, src_r, dst_r, z32)
    h3 = _layer3(*a3, *xs2, dis2d, W3, b3.reshape(1, -1))
    return _pool(batch.reshape(1, N), Wout, bout.reshape(1, -1), *h3)


# speedup vs baseline: 16.1526x; 1.0516x over previous
import functools

import jax
import jax.numpy as jnp
from jax import lax
from jax.experimental import pallas as pl
from jax.experimental.pallas import tpu as pltpu
from jax.experimental.pallas import tpu_sc as plsc

N = 10000
E = 320000
B = 64
NSC = 2
NTIL = 16
NB = 80
K = 125
NBUF = 4
DRT = 624
TOFF = DRT * NTIL
TAIL = N - TOFF
ZR = 208
NP1 = 10240

@functools.cache
def _make_deg():
    return pl.kernel(
        _deg_body,
        out_type=jax.ShapeDtypeStruct((NSC * NP1,), jnp.float32),
        mesh=plsc.VectorSubcoreMesh(core_axis_name="c", subcore_axis_name="s"),
        scratch_types=[
            pltpu.VMEM((NB, K), jnp.int32),
            pltpu.VMEM((NB, K), jnp.float32),
            pltpu.VMEM((1024,), jnp.float32),
            pltpu.VMEM_SHARED((NP1,), jnp.float32),
        ],
    )


def _deg_body(dst_hbm, ones_hbm, z1_hbm, out_hbm, idx_v, ones_v, stage, acc):
    c = lax.axis_index("c")
    s = lax.axis_index("s")
    pltpu.sync_copy(dst_hbm.at[c, s], idx_v)
    pltpu.sync_copy(ones_hbm, ones_v)

    @pl.when(s < 10)
    def _():
        pltpu.sync_copy(z1_hbm, stage)
        pltpu.sync_copy(stage, acc.at[pl.ds(s * 1024, 1024)])

    plsc.subcore_barrier()

    def body(j, carry):
        pltpu.sync_copy(ones_v.at[j], acc.at[idx_v.at[j]], add=True)
        return carry

    lax.fori_loop(0, NB, body, 0)
    plsc.subcore_barrier()

    @pl.when(s < 10)
    def _():
        pltpu.sync_copy(acc.at[pl.ds(s * 1024, 1024)], stage)
        pltpu.sync_copy(stage, out_hbm.at[pl.ds(c * NP1 + s * 1024, 1024)])


@functools.cache
def _make_agg(w, nch, nbuf=NBUF):
    def body(*refs):
        h_list = refs[:nch]
        src_hbm, dst_hbm, z2_hbm = refs[nch:nch + 3]
        out_list = refs[nch + 3:2 * nch + 3]
        (src_v, dst_v, zbuf, obuf, acc) = refs[2 * nch + 3:2 * nch + 8]
        bufs = refs[2 * nch + 8:2 * nch + 8 + nbuf]
        gsem = refs[2 * nch + 8 + nbuf:2 * nch + 8 + 2 * nbuf]
        ssem = refs[2 * nch + 8 + 2 * nbuf:]
        c = lax.axis_index("c")
        s = lax.axis_index("s")
        pltpu.sync_copy(src_hbm.at[c, s], src_v)
        pltpu.sync_copy(dst_hbm.at[c, s], dst_v)
        pltpu.sync_copy(z2_hbm, zbuf)
        for ch in range(nch):
            h_hbm = h_list[ch]
            for k3 in range(3):
                pltpu.sync_copy(zbuf, acc.at[pl.ds(s * DRT + k3 * ZR, ZR)])

            @pl.when(s == NTIL - 1)
            def _():
                pltpu.sync_copy(zbuf.at[pl.ds(0, TAIL)],
                                acc.at[pl.ds(TOFF, TAIL)])

            plsc.subcore_barrier()

            def ebody(t, carry):
                base = t * nbuf
                for b in range(nbuf):
                    @pl.when(t > 0)
                    def _():
                        pltpu.make_async_copy(
                            bufs[b], acc.at[dst_v.at[base - nbuf + b]],
                            ssem[b]).wait()
                    pltpu.async_copy(h_hbm.at[src_v.at[base + b]],
                                     bufs[b], gsem[b])
                for b in range(nbuf):
                    pltpu.make_async_copy(h_hbm.at[src_v.at[base + b]],
                                          bufs[b], gsem[b]).wait()
                    pltpu.async_copy(bufs[b], acc.at[dst_v.at[base + b]],
                                     ssem[b], add=True)
                return carry

            lax.fori_loop(0, NB // nbuf, ebody, 0)
            for b in range(nbuf):
                pltpu.make_async_copy(
                    bufs[b], acc.at[dst_v.at[NB - nbuf + b]], ssem[b]).wait()
            plsc.subcore_barrier()
            for k3 in range(3):
                pltpu.sync_copy(acc.at[pl.ds(s * DRT + k3 * ZR, ZR)], obuf)
                pltpu.sync_copy(
                    obuf, out_list[ch].at[c, pl.ds(s * DRT + k3 * ZR, ZR)])

            @pl.when(s == NTIL - 1)
            def _():
                pltpu.sync_copy(acc.at[pl.ds(TOFF, TAIL)],
                                obuf.at[pl.ds(0, TAIL)])
                pltpu.sync_copy(obuf.at[pl.ds(0, TAIL)],
                                out_list[ch].at[c, pl.ds(TOFF, TAIL)])

    return pl.kernel(
        body,
        out_type=[jax.ShapeDtypeStruct((NSC, N, w), jnp.float32)] * nch,
        mesh=plsc.VectorSubcoreMesh(core_axis_name="c", subcore_axis_name="s"),
        compiler_params=pltpu.CompilerParams(use_tc_tiling_on_sc=False),
        scratch_types=(
            [pltpu.VMEM((NB, K), jnp.int32),
             pltpu.VMEM((NB, K), jnp.int32),
             pltpu.VMEM((ZR, w), jnp.float32),
             pltpu.VMEM((ZR, w), jnp.float32),
             pltpu.VMEM_SHARED((N, w), jnp.float32)]
            + [pltpu.VMEM((K, w), jnp.float32)] * nbuf
            + [pltpu.SemaphoreType.DMA] * (2 * nbuf)
        ),
    )


def _scale_body(dp_ref, x_ref, xs0_ref, xs1_ref, dis_ref):
    deg = dp_ref[...][0] + dp_ref[...][1] + 1.0
    dis = lax.rsqrt(deg)
    dis_ref[...] = jnp.broadcast_to(dis, (N, 128))
    xs = x_ref[...] * dis
    xs0_ref[...] = xs[:, :64]
    xs1_ref[...] = xs[:, 64:]


_scale = pl.pallas_call(
    _scale_body,
    out_shape=(jax.ShapeDtypeStruct((N, 64), jnp.float32),
               jax.ShapeDtypeStruct((N, 64), jnp.float32),
               jax.ShapeDtypeStruct((N, 128), jnp.float32)),
)


def _make_layer(nch_in, w_in, nch_out, w_out, d_out, scale_out, R=1000):
    d_in = nch_in * w_in

    def body(*refs):
        agg_refs = refs[0:nch_in]
        xs_refs = refs[nch_in:2 * nch_in]
        dis_ref = refs[2 * nch_in]
        w_ref = refs[1 + 2 * nch_in]
        b_ref = refs[2 + 2 * nch_in]
        out_refs = refs[3 + 2 * nch_in:]
        aggc = jnp.concatenate(
            [r[...][0] + r[...][1] for r in agg_refs], axis=1)
        xsc = jnp.concatenate([r[...] for r in xs_refs], axis=1)
        dis = dis_ref[...][:, 0:1]
        g = dis * (aggc + xsc)
        h = lax.dot_general(g, w_ref[...], (((1,), (1,)), ((), ())),
                            preferred_element_type=jnp.float32)
        h = jnp.maximum(h + b_ref[...], 0.0)
        if scale_out:
            h = h * dis
        for k2 in range(nch_out):
            out_refs[k2][...] = h[:, k2 * w_out:(k2 + 1) * w_out]

    return pl.pallas_call(
        body,
        grid=(N // R,),
        in_specs=(
            [pl.BlockSpec((NSC, R, w_in), lambda i: (0, i, 0))] * nch_in
            + [pl.BlockSpec((R, w_in), lambda i: (i, 0))] * nch_in
            + [pl.BlockSpec((R, 128), lambda i: (i, 0)),
               pl.BlockSpec((d_out, d_in), lambda i: (0, 0)),
               pl.BlockSpec((1, d_out), lambda i: (0, 0))]),
        out_specs=[pl.BlockSpec((R, w_out), lambda i: (i, 0))] * nch_out,
        out_shape=[jax.ShapeDtypeStruct((N, w_out), jnp.float32)] * nch_out,
    )


_layer1 = _make_layer(2, 64, 4, 64, 256, True)
_layer2 = _make_layer(4, 64, 16, 32, 512, True)
_layer3 = _make_layer(16, 32, 8, 128, 1024, False, R=400)


def _pool_body(*refs):
    b_ref, w_ref, bo_ref = refs[0], refs[1], refs[2]
    h_refs = refs[3:11]
    out_ref = refs[11]
    h3 = jnp.concatenate([r[...] for r in h_refs], axis=1)
    bvec = b_ref[...]
    io = lax.broadcasted_iota(jnp.int32, (B, N), 0)
    oh = (bvec == io).astype(jnp.float32)
    pooled = lax.dot_general(oh, h3, (((1,), (0,)), ((), ())),
                             preferred_element_type=jnp.float32)
    cnt = jnp.sum(oh, axis=1, keepdims=True)
    pooled = pooled / jnp.maximum(cnt, 1.0)
    out = lax.dot_general(pooled, w_ref[...], (((1,), (1,)), ((), ())),
                          preferred_element_type=jnp.float32)
    out_ref[...] = out + bo_ref[...]


_pool = pl.pallas_call(
    _pool_body,
    out_shape=jax.ShapeDtypeStruct((B, 1024), jnp.float32),
)


def kernel(x, edge_index, batch, W1, b1, W2, b2, W3, b3, Wout, bout):
    src_r = edge_index[0].reshape(NSC, NTIL, NB, K)
    dst_r = edge_index[1].reshape(NSC, NTIL, NB, K)
    ones_upd = jnp.ones((NB, K), jnp.float32)
    z1 = jnp.zeros((1024,), jnp.float32)
    z64 = jnp.zeros((ZR, 64), jnp.float32)
    z32 = jnp.zeros((ZR, 32), jnp.float32)

    degp = _make_deg()(dst_r, ones_upd, z1)
    degp = degp.reshape(NSC, NP1)[:, :N].reshape(NSC, N, 1)
    xs0a, xs0b, dis2d = _scale(degp, x)
    a1 = _make_agg(64, 2)(xs0a, xs0b, src_r, dst_r, z64)
    xs1 = _layer1(*a1, xs0a, xs0b, dis2d, W1, b1.reshape(1, -1))
    a2 = _make_agg(64, 4)(*xs1, src_r, dst_r, z64)
    xs2 = _layer2(*a2, *xs1, dis2d, W2, b2.reshape(1, -1))
    a3 = _make_agg(32, 16, 8)(*xs2, src_r, dst_r, z32)
    h3 = _layer3(*a3, *xs2, dis2d, W3, b3.reshape(1, -1))
    return _pool(batch.reshape(1, N), Wout, bout.reshape(1, -1), *h3)


# speedup vs baseline: 16.6252x; 1.0293x over previous
import functools

import jax
import jax.numpy as jnp
from jax import lax
from jax.experimental import pallas as pl
from jax.experimental.pallas import tpu as pltpu
from jax.experimental.pallas import tpu_sc as plsc

N = 10000
E = 320000
B = 64
NSC = 2
NTIL = 16
NB = 80
K = 125
NBUF = 4
DRT = 624
TOFF = DRT * NTIL
TAIL = N - TOFF
ZR = 208
NP1 = 10240

@functools.cache
def _make_deg():
    return pl.kernel(
        _deg_body,
        out_type=jax.ShapeDtypeStruct((NSC * NP1,), jnp.float32),
        mesh=plsc.VectorSubcoreMesh(core_axis_name="c", subcore_axis_name="s"),
        scratch_types=[
            pltpu.VMEM((NB, K), jnp.int32),
            pltpu.VMEM((NB, K), jnp.float32),
            pltpu.VMEM((1024,), jnp.float32),
            pltpu.VMEM_SHARED((NP1,), jnp.float32),
        ],
    )


def _deg_body(dst_hbm, ones_hbm, z1_hbm, out_hbm, idx_v, ones_v, stage, acc):
    c = lax.axis_index("c")
    s = lax.axis_index("s")
    pltpu.sync_copy(dst_hbm.at[c, s], idx_v)
    pltpu.sync_copy(ones_hbm, ones_v)

    @pl.when(s < 10)
    def _():
        pltpu.sync_copy(z1_hbm, stage)
        pltpu.sync_copy(stage, acc.at[pl.ds(s * 1024, 1024)])

    plsc.subcore_barrier()

    def body(j, carry):
        pltpu.sync_copy(ones_v.at[j], acc.at[idx_v.at[j]], add=True)
        return carry

    lax.fori_loop(0, NB, body, 0)
    plsc.subcore_barrier()

    @pl.when(s < 10)
    def _():
        pltpu.sync_copy(acc.at[pl.ds(s * 1024, 1024)], stage)
        pltpu.sync_copy(stage, out_hbm.at[pl.ds(c * NP1 + s * 1024, 1024)])


@functools.cache
def _make_agg(w, nch, nbuf=NBUF):
    def body(*refs):
        h_list = refs[:nch]
        src_hbm, dst_hbm, z2_hbm = refs[nch:nch + 3]
        out_list = refs[nch + 3:2 * nch + 3]
        (src_v, dst_v, zbuf, obuf, acc) = refs[2 * nch + 3:2 * nch + 8]
        bufs = refs[2 * nch + 8:2 * nch + 8 + nbuf]
        gsem = refs[2 * nch + 8 + nbuf:2 * nch + 8 + 2 * nbuf]
        ssem = refs[2 * nch + 8 + 2 * nbuf:]
        c = lax.axis_index("c")
        s = lax.axis_index("s")
        pltpu.sync_copy(src_hbm.at[c, s], src_v)
        pltpu.sync_copy(dst_hbm.at[c, s], dst_v)
        pltpu.sync_copy(z2_hbm, zbuf)
        for ch in range(nch):
            h_hbm = h_list[ch]
            for k3 in range(3):
                pltpu.sync_copy(zbuf, acc.at[pl.ds(s * DRT + k3 * ZR, ZR)])

            @pl.when(s == NTIL - 1)
            def _():
                pltpu.sync_copy(zbuf.at[pl.ds(0, TAIL)],
                                acc.at[pl.ds(TOFF, TAIL)])

            plsc.subcore_barrier()

            def ebody(t, carry):
                base = t * nbuf
                for b in range(nbuf):
                    @pl.when(t > 0)
                    def _():
                        pltpu.make_async_copy(
                            bufs[b], acc.at[dst_v.at[base - nbuf + b]],
                            ssem[b]).wait()
                    pltpu.async_copy(h_hbm.at[src_v.at[base + b]],
                                     bufs[b], gsem[b])
                for b in range(nbuf):
                    pltpu.make_async_copy(h_hbm.at[src_v.at[base + b]],
                                          bufs[b], gsem[b]).wait()
                    pltpu.async_copy(bufs[b], acc.at[dst_v.at[base + b]],
                                     ssem[b], add=True)
                return carry

            lax.fori_loop(0, NB // nbuf, ebody, 0)
            for b in range(nbuf):
                pltpu.make_async_copy(
                    bufs[b], acc.at[dst_v.at[NB - nbuf + b]], ssem[b]).wait()
            plsc.subcore_barrier()
            for k3 in range(3):
                pltpu.sync_copy(acc.at[pl.ds(s * DRT + k3 * ZR, ZR)], obuf)
                pltpu.sync_copy(
                    obuf, out_list[ch].at[c, pl.ds(s * DRT + k3 * ZR, ZR)])

            @pl.when(s == NTIL - 1)
            def _():
                pltpu.sync_copy(acc.at[pl.ds(TOFF, TAIL)],
                                obuf.at[pl.ds(0, TAIL)])
                pltpu.sync_copy(obuf.at[pl.ds(0, TAIL)],
                                out_list[ch].at[c, pl.ds(TOFF, TAIL)])

    return pl.kernel(
        body,
        out_type=[jax.ShapeDtypeStruct((NSC, N, w), jnp.float32)] * nch,
        mesh=plsc.VectorSubcoreMesh(core_axis_name="c", subcore_axis_name="s"),
        compiler_params=pltpu.CompilerParams(use_tc_tiling_on_sc=False),
        scratch_types=(
            [pltpu.VMEM((NB, K), jnp.int32),
             pltpu.VMEM((NB, K), jnp.int32),
             pltpu.VMEM((ZR, w), jnp.float32),
             pltpu.VMEM((ZR, w), jnp.float32),
             pltpu.VMEM_SHARED((N, w), jnp.float32)]
            + [pltpu.VMEM((K, w), jnp.float32)] * nbuf
            + [pltpu.SemaphoreType.DMA] * (2 * nbuf)
        ),
    )


def _scale_body(dp_ref, x_ref, xs0_ref, xs1_ref, dis_ref):
    deg = dp_ref[...][0] + dp_ref[...][1] + 1.0
    dis = lax.rsqrt(deg)
    dis_ref[...] = jnp.broadcast_to(dis, (N, 128))
    xs = x_ref[...] * dis
    xs0_ref[...] = xs[:, :64]
    xs1_ref[...] = xs[:, 64:]


_scale = pl.pallas_call(
    _scale_body,
    out_shape=(jax.ShapeDtypeStruct((N, 64), jnp.float32),
               jax.ShapeDtypeStruct((N, 64), jnp.float32),
               jax.ShapeDtypeStruct((N, 128), jnp.float32)),
)


def _make_layer(nch_in, w_in, nch_out, w_out, d_out, scale_out, R=1000):
    d_in = nch_in * w_in

    def body(*refs):
        agg_refs = refs[0:nch_in]
        xs_refs = refs[nch_in:2 * nch_in]
        dis_ref = refs[2 * nch_in]
        w_ref = refs[1 + 2 * nch_in]
        b_ref = refs[2 + 2 * nch_in]
        out_refs = refs[3 + 2 * nch_in:]
        aggc = jnp.concatenate(
            [r[...][0] + r[...][1] for r in agg_refs], axis=1)
        xsc = jnp.concatenate([r[...] for r in xs_refs], axis=1)
        dis = dis_ref[...][:, 0:1]
        g = dis * (aggc + xsc)
        h = lax.dot_general(g, w_ref[...], (((1,), (1,)), ((), ())),
                            preferred_element_type=jnp.float32)
        h = jnp.maximum(h + b_ref[...], 0.0)
        if scale_out:
            h = h * dis
        for k2 in range(nch_out):
            out_refs[k2][...] = h[:, k2 * w_out:(k2 + 1) * w_out]

    return pl.pallas_call(
        body,
        grid=(N // R,),
        in_specs=(
            [pl.BlockSpec((NSC, R, w_in), lambda i: (0, i, 0))] * nch_in
            + [pl.BlockSpec((R, w_in), lambda i: (i, 0))] * nch_in
            + [pl.BlockSpec((R, 128), lambda i: (i, 0)),
               pl.BlockSpec((d_out, d_in), lambda i: (0, 0)),
               pl.BlockSpec((1, d_out), lambda i: (0, 0))]),
        out_specs=[pl.BlockSpec((R, w_out), lambda i: (i, 0))] * nch_out,
        out_shape=[jax.ShapeDtypeStruct((N, w_out), jnp.float32)] * nch_out,
    )


_layer1 = _make_layer(2, 64, 4, 64, 256, True)
_layer2 = _make_layer(4, 64, 16, 32, 512, True)


def _l3pool_body(*refs):
    nch_in = 16
    R3 = 400
    agg_refs = refs[0:nch_in]
    xs_refs = refs[nch_in:2 * nch_in]
    dis_ref = refs[2 * nch_in]
    w_ref = refs[1 + 2 * nch_in]
    b_ref = refs[2 + 2 * nch_in]
    bt_ref = refs[3 + 2 * nch_in]
    wo_ref = refs[4 + 2 * nch_in]
    bo_ref = refs[5 + 2 * nch_in]
    out_ref = refs[6 + 2 * nch_in]
    ps_ref = refs[7 + 2 * nch_in]
    cnt_ref = refs[8 + 2 * nch_in]
    i = pl.program_id(0)
    aggc = jnp.concatenate([r[...][0] + r[...][1] for r in agg_refs], axis=1)
    xsc = jnp.concatenate([r[...] for r in xs_refs], axis=1)
    dis = dis_ref[...][:, 0:1]
    g = dis * (aggc + xsc)
    h = lax.dot_general(g, w_ref[...], (((1,), (1,)), ((), ())),
                        preferred_element_type=jnp.float32)
    h = jnp.maximum(h + b_ref[...], 0.0)
    bvec = bt_ref[...][0]
    io = lax.broadcasted_iota(jnp.int32, (B, R3), 0)
    oh = (bvec == io).astype(jnp.float32)
    psum = lax.dot_general(oh, h, (((1,), (0,)), ((), ())),
                           preferred_element_type=jnp.float32)
    csum = jnp.broadcast_to(jnp.sum(oh, axis=1, keepdims=True), (B, 128))

    @pl.when(i == 0)
    def _():
        ps_ref[...] = psum
        cnt_ref[...] = csum

    @pl.when(i > 0)
    def _():
        ps_ref[...] += psum
        cnt_ref[...] += csum

    @pl.when(i == N // R3 - 1)
    def _():
        pooled = ps_ref[...] / jnp.maximum(cnt_ref[...][:, 0:1], 1.0)
        out_ref[...] = lax.dot_general(
            pooled, wo_ref[...], (((1,), (1,)), ((), ())),
            preferred_element_type=jnp.float32) + bo_ref[...]


def _make_l3pool():
    R3 = 400
    return pl.pallas_call(
        _l3pool_body,
        grid=(N // R3,),
        in_specs=(
            [pl.BlockSpec((NSC, R3, 32), lambda i: (0, i, 0))] * 16
            + [pl.BlockSpec((R3, 32), lambda i: (i, 0))] * 16
            + [pl.BlockSpec((R3, 128), lambda i: (i, 0)),
               pl.BlockSpec((1024, 512), lambda i: (0, 0)),
               pl.BlockSpec((1, 1024), lambda i: (0, 0)),
               pl.BlockSpec((1, 1, R3), lambda i: (i, 0, 0)),
               pl.BlockSpec((1024, 1024), lambda i: (0, 0)),
               pl.BlockSpec((1, 1024), lambda i: (0, 0))]),
        out_specs=pl.BlockSpec((B, 1024), lambda i: (0, 0)),
        out_shape=jax.ShapeDtypeStruct((B, 1024), jnp.float32),
        scratch_shapes=[pltpu.VMEM((B, 1024), jnp.float32),
                        pltpu.VMEM((B, 128), jnp.float32)],
    )


_l3pool = _make_l3pool()


def kernel(x, edge_index, batch, W1, b1, W2, b2, W3, b3, Wout, bout):
    src_r = edge_index[0].reshape(NSC, NTIL, NB, K)
    dst_r = edge_index[1].reshape(NSC, NTIL, NB, K)
    ones_upd = jnp.ones((NB, K), jnp.float32)
    z1 = jnp.zeros((1024,), jnp.float32)
    z64 = jnp.zeros((ZR, 64), jnp.float32)
    z32 = jnp.zeros((ZR, 32), jnp.float32)

    degp = _make_deg()(dst_r, ones_upd, z1)
    degp = degp.reshape(NSC, NP1)[:, :N].reshape(NSC, N, 1)
    xs0a, xs0b, dis2d = _scale(degp, x)
    a1 = _make_agg(64, 2, 5)(xs0a, xs0b, src_r, dst_r, z64)
    xs1 = _layer1(*a1, xs0a, xs0b, dis2d, W1, b1.reshape(1, -1))
    a2 = _make_agg(64, 4, 5)(*xs1, src_r, dst_r, z64)
    xs2 = _layer2(*a2, *xs1, dis2d, W2, b2.reshape(1, -1))
    a3 = _make_agg(32, 16, 8)(*xs2, src_r, dst_r, z32)
    return _l3pool(*a3, *xs2, dis2d, W3, b3.reshape(1, -1),
                   batch.reshape(25, 1, 400), Wout, bout.reshape(1, -1))


# speedup vs baseline: 16.6536x; 1.0017x over previous
import functools

import jax
import jax.numpy as jnp
from jax import lax
from jax.experimental import pallas as pl
from jax.experimental.pallas import tpu as pltpu
from jax.experimental.pallas import tpu_sc as plsc

N = 10000
E = 320000
B = 64
NSC = 2
NTIL = 16
NB = 80
K = 125
NBUF = 4
DRT = 624
TOFF = DRT * NTIL
TAIL = N - TOFF
ZR = 208
NP1 = 10240

@functools.cache
def _make_deg():
    return pl.kernel(
        _deg_body,
        out_type=jax.ShapeDtypeStruct((NSC * NP1,), jnp.float32),
        mesh=plsc.VectorSubcoreMesh(core_axis_name="c", subcore_axis_name="s"),
        scratch_types=[
            pltpu.VMEM((NB, K), jnp.int32),
            pltpu.VMEM((NB, K), jnp.float32),
            pltpu.VMEM((1024,), jnp.float32),
            pltpu.VMEM_SHARED((NP1,), jnp.float32),
        ],
    )


def _deg_body(dst_hbm, ones_hbm, z1_hbm, out_hbm, idx_v, ones_v, stage, acc):
    c = lax.axis_index("c")
    s = lax.axis_index("s")
    pltpu.sync_copy(dst_hbm.at[c, s], idx_v)
    pltpu.sync_copy(ones_hbm, ones_v)

    @pl.when(s < 10)
    def _():
        pltpu.sync_copy(z1_hbm, stage)
        pltpu.sync_copy(stage, acc.at[pl.ds(s * 1024, 1024)])

    plsc.subcore_barrier()

    def body(j, carry):
        pltpu.sync_copy(ones_v.at[j], acc.at[idx_v.at[j]], add=True)
        return carry

    lax.fori_loop(0, NB, body, 0)
    plsc.subcore_barrier()

    @pl.when(s < 10)
    def _():
        pltpu.sync_copy(acc.at[pl.ds(s * 1024, 1024)], stage)
        pltpu.sync_copy(stage, out_hbm.at[pl.ds(c * NP1 + s * 1024, 1024)])


@functools.cache
def _make_agg(w, nch, nbuf=NBUF):
    def body(*refs):
        (h3d, src_hbm, dst_hbm, z2_hbm, out_hbm,
         src_v, dst_v, zbuf, obuf, acc) = refs[:10]
        bufs = refs[10:10 + nbuf]
        gsem = refs[10 + nbuf:10 + 2 * nbuf]
        ssem = refs[10 + 2 * nbuf:]
        c = lax.axis_index("c")
        s = lax.axis_index("s")
        pltpu.sync_copy(src_hbm.at[c, s], src_v)
        pltpu.sync_copy(dst_hbm.at[c, s], dst_v)
        pltpu.sync_copy(z2_hbm, zbuf)
        for ch in range(nch):
            h_hbm = h3d.at[ch]
            for k3 in range(3):
                pltpu.sync_copy(zbuf, acc.at[pl.ds(s * DRT + k3 * ZR, ZR)])

            @pl.when(s == NTIL - 1)
            def _():
                pltpu.sync_copy(zbuf.at[pl.ds(0, TAIL)],
                                acc.at[pl.ds(TOFF, TAIL)])

            plsc.subcore_barrier()

            def ebody(t, carry):
                base = t * nbuf
                for b in range(nbuf):
                    @pl.when(t > 0)
                    def _():
                        pltpu.make_async_copy(
                            bufs[b], acc.at[dst_v.at[base - nbuf + b]],
                            ssem[b]).wait()
                    pltpu.async_copy(h_hbm.at[src_v.at[base + b]],
                                     bufs[b], gsem[b])
                for b in range(nbuf):
                    pltpu.make_async_copy(h_hbm.at[src_v.at[base + b]],
                                          bufs[b], gsem[b]).wait()
                    pltpu.async_copy(bufs[b], acc.at[dst_v.at[base + b]],
                                     ssem[b], add=True)
                return carry

            lax.fori_loop(0, NB // nbuf, ebody, 0)
            for b in range(nbuf):
                pltpu.make_async_copy(
                    bufs[b], acc.at[dst_v.at[NB - nbuf + b]], ssem[b]).wait()
            plsc.subcore_barrier()
            for k3 in range(3):
                pltpu.sync_copy(acc.at[pl.ds(s * DRT + k3 * ZR, ZR)], obuf)
                pltpu.sync_copy(
                    obuf, out_hbm.at[c, ch, pl.ds(s * DRT + k3 * ZR, ZR)])

            @pl.when(s == NTIL - 1)
            def _():
                pltpu.sync_copy(acc.at[pl.ds(TOFF, TAIL)],
                                obuf.at[pl.ds(0, TAIL)])
                pltpu.sync_copy(obuf.at[pl.ds(0, TAIL)],
                                out_hbm.at[c, ch, pl.ds(TOFF, TAIL)])

    return pl.kernel(
        body,
        out_type=jax.ShapeDtypeStruct((NSC, nch, N, w), jnp.float32),
        mesh=plsc.VectorSubcoreMesh(core_axis_name="c", subcore_axis_name="s"),
        compiler_params=pltpu.CompilerParams(use_tc_tiling_on_sc=False),
        scratch_types=(
            [pltpu.VMEM((NB, K), jnp.int32),
             pltpu.VMEM((NB, K), jnp.int32),
             pltpu.VMEM((ZR, w), jnp.float32),
             pltpu.VMEM((ZR, w), jnp.float32),
             pltpu.VMEM_SHARED((N, w), jnp.float32)]
            + [pltpu.VMEM((K, w), jnp.float32)] * nbuf
            + [pltpu.SemaphoreType.DMA] * (2 * nbuf)
        ),
    )


def _scale_body(dp_ref, x_ref, xs_ref, dis_ref):
    deg = dp_ref[...][0] + dp_ref[...][1] + 1.0
    dis = lax.rsqrt(deg)
    dis_ref[...] = jnp.broadcast_to(dis, (N, 128))
    xs = x_ref[...] * dis
    xs_ref[...] = jnp.stack([xs[:, :64], xs[:, 64:]], axis=0)


_scale = pl.pallas_call(
    _scale_body,
    out_shape=(jax.ShapeDtypeStruct((2, N, 64), jnp.float32),
               jax.ShapeDtypeStruct((N, 128), jnp.float32)),
)


def _make_layer(nch_in, w_in, nch_out, w_out, d_out, scale_out, R=1000):
    d_in = nch_in * w_in

    def body(agg_ref, xs_ref, dis_ref, w_ref, b_ref, out_ref):
        parts = agg_ref[...]
        agg = parts[0] + parts[1]
        xs = xs_ref[...]
        aggc = jnp.concatenate(
            [agg[ch] + xs[ch] for ch in range(nch_in)], axis=1)
        dis = dis_ref[...][:, 0:1]
        g = dis * aggc
        h = lax.dot_general(g, w_ref[...], (((1,), (1,)), ((), ())),
                            preferred_element_type=jnp.float32)
        h = jnp.maximum(h + b_ref[...], 0.0)
        if scale_out:
            h = h * dis
        out_ref[...] = jnp.stack(
            [h[:, k2 * w_out:(k2 + 1) * w_out] for k2 in range(nch_out)],
            axis=0)

    return pl.pallas_call(
        body,
        grid=(N // R,),
        in_specs=(
            [pl.BlockSpec((NSC, nch_in, R, w_in), lambda i: (0, 0, i, 0)),
             pl.BlockSpec((nch_in, R, w_in), lambda i: (0, i, 0)),
             pl.BlockSpec((R, 128), lambda i: (i, 0)),
             pl.BlockSpec((d_out, d_in), lambda i: (0, 0)),
             pl.BlockSpec((1, d_out), lambda i: (0, 0))]),
        out_specs=pl.BlockSpec((nch_out, R, w_out), lambda i: (0, i, 0)),
        out_shape=jax.ShapeDtypeStruct((nch_out, N, w_out), jnp.float32),
    )


_layer1 = _make_layer(2, 64, 4, 64, 256, True)
_layer2 = _make_layer(4, 64, 16, 32, 512, True)


def _l3pool_body(agg_ref, xs_ref, dis_ref, w_ref, b_ref, bt_ref, wo_ref,
                 bo_ref, out_ref, ps_ref, cnt_ref):
    nch_in = 16
    R3 = 400
    i = pl.program_id(0)
    parts = agg_ref[...]
    agg = parts[0] + parts[1]
    xs = xs_ref[...]
    aggc = jnp.concatenate(
        [agg[ch] + xs[ch] for ch in range(nch_in)], axis=1)
    dis = dis_ref[...][:, 0:1]
    g = dis * aggc
    h = lax.dot_general(g, w_ref[...], (((1,), (1,)), ((), ())),
                        preferred_element_type=jnp.float32)
    h = jnp.maximum(h + b_ref[...], 0.0)
    bvec = bt_ref[...][0]
    io = lax.broadcasted_iota(jnp.int32, (B, R3), 0)
    oh = (bvec == io).astype(jnp.float32)
    psum = lax.dot_general(oh, h, (((1,), (0,)), ((), ())),
                           preferred_element_type=jnp.float32)
    csum = jnp.broadcast_to(jnp.sum(oh, axis=1, keepdims=True), (B, 128))

    @pl.when(i == 0)
    def _():
        ps_ref[...] = psum
        cnt_ref[...] = csum

    @pl.when(i > 0)
    def _():
        ps_ref[...] += psum
        cnt_ref[...] += csum

    @pl.when(i == N // R3 - 1)
    def _():
        pooled = ps_ref[...] / jnp.maximum(cnt_ref[...][:, 0:1], 1.0)
        out_ref[...] = lax.dot_general(
            pooled, wo_ref[...], (((1,), (1,)), ((), ())),
            preferred_element_type=jnp.float32) + bo_ref[...]


def _make_l3pool():
    R3 = 400
    return pl.pallas_call(
        _l3pool_body,
        grid=(N // R3,),
        in_specs=(
            [pl.BlockSpec((NSC, 16, R3, 32), lambda i: (0, 0, i, 0)),
             pl.BlockSpec((16, R3, 32), lambda i: (0, i, 0)),
             pl.BlockSpec((R3, 128), lambda i: (i, 0)),
               pl.BlockSpec((1024, 512), lambda i: (0, 0)),
               pl.BlockSpec((1, 1024), lambda i: (0, 0)),
               pl.BlockSpec((1, 1, R3), lambda i: (i, 0, 0)),
               pl.BlockSpec((1024, 1024), lambda i: (0, 0)),
               pl.BlockSpec((1, 1024), lambda i: (0, 0))]),
        out_specs=pl.BlockSpec((B, 1024), lambda i: (0, 0)),
        out_shape=jax.ShapeDtypeStruct((B, 1024), jnp.float32),
        scratch_shapes=[pltpu.VMEM((B, 1024), jnp.float32),
                        pltpu.VMEM((B, 128), jnp.float32)],
    )


_l3pool = _make_l3pool()


def kernel(x, edge_index, batch, W1, b1, W2, b2, W3, b3, Wout, bout):
    src_r = edge_index[0].reshape(NSC, NTIL, NB, K)
    dst_r = edge_index[1].reshape(NSC, NTIL, NB, K)
    ones_upd = jnp.ones((NB, K), jnp.float32)
    z1 = jnp.zeros((1024,), jnp.float32)
    z64 = jnp.zeros((ZR, 64), jnp.float32)
    z32 = jnp.zeros((ZR, 32), jnp.float32)

    degp = _make_deg()(dst_r, ones_upd, z1)
    degp = degp.reshape(NSC, NP1)[:, :N].reshape(NSC, N, 1)
    xs0, dis2d = _scale(degp, x)
    a1 = _make_agg(64, 2, 5)(xs0, src_r, dst_r, z64)
    xs1 = _layer1(a1, xs0, dis2d, W1, b1.reshape(1, -1))
    a2 = _make_agg(64, 4, 5)(xs1, src_r, dst_r, z64)
    xs2 = _layer2(a2, xs1, dis2d, W2, b2.reshape(1, -1))
    a3 = _make_agg(32, 16, 8)(xs2, src_r, dst_r, z32)
    return _l3pool(a3, xs2, dis2d, W3, b3.reshape(1, -1),
                   batch.reshape(25, 1, 400), Wout, bout.reshape(1, -1))
